# Initial kernel scaffold; baseline (speedup 1.0000x reference)
import functools

import jax
import jax.numpy as jnp
from jax.experimental import pallas as pl
from jax.experimental.pallas import tpu as pltpu


def _scatter_body(paux_ref, k_ref, v_ref, ok_ref, ov_ref, *, bs):
    base = pl.program_id(1) * bs
    tgt = paux_ref[0:1, :]
    nxt = paux_ref[1:2, :]
    is_last = tgt != nxt
    rows = jax.lax.broadcasted_iota(jnp.int32, (bs, tgt.shape[1]), 0) + base
    onehot = ((rows == tgt) & is_last).astype(jnp.float32)
    ok_ref[0] = jnp.dot(onehot, k_ref[0], preferred_element_type=jnp.float32)
    ov_ref[0] = jnp.dot(onehot, v_ref[0], preferred_element_type=jnp.float32)


def _scatter_full(paux, k2, v2, S, *, bs=512):
    B, P, HD = k2.shape
    grid = (B, S // bs)
    return pl.pallas_call(
        functools.partial(_scatter_body, bs=bs),
        grid=grid,
        in_specs=[
            pl.BlockSpec((8, P), lambda b, s: (0, 0)),
            pl.BlockSpec((1, P, HD), lambda b, s: (b, 0, 0)),
            pl.BlockSpec((1, P, HD), lambda b, s: (b, 0, 0)),
        ],
        out_specs=[
            pl.BlockSpec((1, bs, HD), lambda b, s: (b, s, 0)),
            pl.BlockSpec((1, bs, HD), lambda b, s: (b, s, 0)),
        ],
        out_shape=[jax.ShapeDtypeStruct((B, S, HD), jnp.float32)] * 2,
        compiler_params=pltpu.CompilerParams(
            dimension_semantics=("parallel", "parallel"),
        ),
    )(paux, k2, v2)


def kernel(k, v, pos, max_pos, k_cache, v_cache):
    B, P, H, D = k.shape
    S = k_cache.shape[1]
    HD = H * D
    pos_i = pos.astype(jnp.int32) % S
    r = (jnp.asarray(max_pos, jnp.int32) + 1) % S
    pos_adj = (pos_i - r) % S
    nxt = jnp.concatenate([pos_adj[1:], jnp.full((1,), -1, jnp.int32)])
    paux = jnp.zeros((8, P), jnp.int32).at[0].set(pos_adj).at[1].set(nxt)
    ok, ov = _scatter_full(paux, k.reshape(B, P, HD), v.reshape(B, P, HD), S)
    return ok.reshape(B, S, H, D), ov.reshape(B, S, H, D)


# speedup vs baseline: 1.9360x; 1.9360x over previous
import functools

import jax
import jax.numpy as jnp
from jax.experimental import pallas as pl
from jax.experimental.pallas import tpu as pltpu


def _scatter_body(paux_ref, k_ref, v_ref, ok_ref, ov_ref, *, bs):
    base = pl.program_id(1) * bs
    tgt = paux_ref[0:1, :]
    nxt = paux_ref[1:2, :]
    is_last = tgt != nxt
    rows = jax.lax.broadcasted_iota(jnp.int32, (bs, tgt.shape[1]), 0) + base
    onehot = ((rows == tgt) & is_last).astype(jnp.float32)
    ok_ref[0] = jnp.dot(onehot, k_ref[0], preferred_element_type=jnp.float32,
                        precision=jax.lax.Precision.HIGHEST)
    ov_ref[0] = jnp.dot(onehot, v_ref[0], preferred_element_type=jnp.float32,
                        precision=jax.lax.Precision.HIGHEST)


def _scatter_full(paux, k2, v2, S, *, bs=512):
    B, P, HD = k2.shape
    grid = (B, S // bs)
    return pl.pallas_call(
        functools.partial(_scatter_body, bs=bs),
        grid=grid,
        in_specs=[
            pl.BlockSpec((8, P), lambda b, s: (0, 0)),
            pl.BlockSpec((1, P, HD), lambda b, s: (b, 0, 0)),
            pl.BlockSpec((1, P, HD), lambda b, s: (b, 0, 0)),
        ],
        out_specs=[
            pl.BlockSpec((1, bs, HD), lambda b, s: (b, s, 0)),
            pl.BlockSpec((1, bs, HD), lambda b, s: (b, s, 0)),
        ],
        out_shape=[jax.ShapeDtypeStruct((B, S, HD), jnp.float32)] * 2,
        compiler_params=pltpu.CompilerParams(
            dimension_semantics=("parallel", "parallel"),
        ),
    )(paux, k2, v2)


def kernel(k, v, pos, max_pos, k_cache, v_cache):
    B, P, H, D = k.shape
    S = k_cache.shape[1]
    HD = H * D
    pos_i = pos.astype(jnp.int32) % S
    r = (jnp.asarray(max_pos, jnp.int32) + 1) % S
    pos_adj = (pos_i - r) % S
    nxt = jnp.concatenate([pos_adj[1:], jnp.full((1,), -1, jnp.int32)])
    paux = jnp.zeros((8, P), jnp.int32).at[0].set(pos_adj).at[1].set(nxt)
    ok, ov = _scatter_full(paux, k.reshape(B, P, HD), v.reshape(B, P, HD), S)
    return ok.reshape(B, S, H, D), ov.reshape(B, S, H, D)


# speedup vs baseline: 2.5739x; 1.3295x over previous
import functools

import jax
import jax.numpy as jnp
from jax.experimental import pallas as pl
from jax.experimental.pallas import tpu as pltpu


def _scatter_body(pos_ref, k_ref, v_ref, ok_ref, ov_ref, *, bs, P):
    base = pl.program_id(1) * bs
    ok_ref[...] = jnp.zeros_like(ok_ref)
    ov_ref[...] = jnp.zeros_like(ov_ref)

    def step(p, c):
        t = pos_ref[p] - base

        @pl.when((t >= 0) & (t < bs))
        def _():
            ok_ref[0, pl.ds(t, 1), :] = k_ref[0, pl.ds(p, 1), :]
            ov_ref[0, pl.ds(t, 1), :] = v_ref[0, pl.ds(p, 1), :]

        return c

    jax.lax.fori_loop(0, P, step, 0)


def _scatter_full(pos_adj, k2, v2, S, *, bs=512):
    B, P, HD = k2.shape
    grid = (B, S // bs)
    return pl.pallas_call(
        functools.partial(_scatter_body, bs=bs, P=P),
        grid_spec=pltpu.PrefetchScalarGridSpec(
            num_scalar_prefetch=1,
            grid=grid,
            in_specs=[
                pl.BlockSpec((1, P, HD), lambda b, s, pref: (b, 0, 0)),
                pl.BlockSpec((1, P, HD), lambda b, s, pref: (b, 0, 0)),
            ],
            out_specs=[
                pl.BlockSpec((1, bs, HD), lambda b, s, pref: (b, s, 0)),
                pl.BlockSpec((1, bs, HD), lambda b, s, pref: (b, s, 0)),
            ],
        ),
        out_shape=[jax.ShapeDtypeStruct((B, S, HD), jnp.float32)] * 2,
        compiler_params=pltpu.CompilerParams(
            dimension_semantics=("parallel", "parallel"),
        ),
    )(pos_adj, k2, v2)


def kernel(k, v, pos, max_pos, k_cache, v_cache):
    B, P, H, D = k.shape
    S = k_cache.shape[1]
    HD = H * D
    pos_i = pos.astype(jnp.int32) % S
    r = (jnp.asarray(max_pos, jnp.int32) + 1) % S
    pos_adj = (pos_i - r) % S
    ok, ov = _scatter_full(pos_adj, k.reshape(B, P, HD), v.reshape(B, P, HD), S)
    return ok.reshape(B, S, H, D), ov.reshape(B, S, H, D)


# speedup vs baseline: 2.6910x; 1.0455x over previous
import functools

import jax
import jax.numpy as jnp
from jax.experimental import pallas as pl
from jax.experimental.pallas import tpu as pltpu


def _scatter_body(pos_ref, k_ref, v_ref, ok_ref, ov_ref, *, bs, P):
    base = pl.program_id(1) * bs
    ok_ref[...] = jnp.zeros_like(ok_ref)
    ov_ref[...] = jnp.zeros_like(ov_ref)

    def step(p, c):
        t = pos_ref[p] - base

        @pl.when((t >= 0) & (t < bs))
        def _():
            ok_ref[0, pl.ds(t, 1), :] = k_ref[0, pl.ds(p, 1), :]
            ov_ref[0, pl.ds(t, 1), :] = v_ref[0, pl.ds(p, 1), :]

        return c

    jax.lax.fori_loop(0, P, step, 0)


def _scatter_full(pos_adj, k2, v2, S, *, bs=1024):
    B, P, HD = k2.shape
    grid = (B, S // bs)
    return pl.pallas_call(
        functools.partial(_scatter_body, bs=bs, P=P),
        grid_spec=pltpu.PrefetchScalarGridSpec(
            num_scalar_prefetch=1,
            grid=grid,
            in_specs=[
                pl.BlockSpec((1, P, HD), lambda b, s, pref: (b, 0, 0)),
                pl.BlockSpec((1, P, HD), lambda b, s, pref: (b, 0, 0)),
            ],
            out_specs=[
                pl.BlockSpec((1, bs, HD), lambda b, s, pref: (b, s, 0)),
                pl.BlockSpec((1, bs, HD), lambda b, s, pref: (b, s, 0)),
            ],
        ),
        out_shape=[jax.ShapeDtypeStruct((B, S, HD), jnp.float32)] * 2,
        compiler_params=pltpu.CompilerParams(
            dimension_semantics=("parallel", "parallel"),
        ),
    )(pos_adj, k2, v2)


def kernel(k, v, pos, max_pos, k_cache, v_cache):
    B, P, H, D = k.shape
    S = k_cache.shape[1]
    HD = H * D
    pos_i = pos.astype(jnp.int32) % S
    r = (jnp.asarray(max_pos, jnp.int32) + 1) % S
    pos_adj = (pos_i - r) % S
    ok, ov = _scatter_full(pos_adj, k.reshape(B, P, HD), v.reshape(B, P, HD), S)
    return ok.reshape(B, S, H, D), ov.reshape(B, S, H, D)
